# Initial kernel scaffold; baseline (speedup 1.0000x reference)
import jax
import jax.numpy as jnp
from jax.experimental import pallas as pl


def kernel(queries, keys, values):
    raise NotImplementedError("write your pallas kernel here")


# speedup vs baseline: 3.9544x; 3.9544x over previous
import functools

import jax
import jax.numpy as jnp
import numpy as np
from jax.experimental import pallas as pl
from jax.experimental.pallas import tpu as pltpu

_B, _L, _H, _D = 2, 2048, 16, 64
_BH = _B * _H
_U = 40
_QBLK = 512
_CBLK = 256


def _count_matrix_T():
    cpu = jax.devices("cpu")[0]
    with jax.default_device(cpu):
        skey = jax.random.key(42)
        idx = jax.random.randint(skey, (_L, _U), 0, _L)
        idx_np = np.asarray(idx)
    ct = np.zeros((_L, _L), dtype=np.float32)
    np.add.at(ct, (idx_np.reshape(-1), np.repeat(np.arange(_L), _U)), 1.0)
    return ct


_CT_NP = _count_matrix_T()


def _body(q_ref, k_ref, v_ref, ct_ref, o_ref, qr_ref, thr_ref, idx_ref):
    K = k_ref[0]
    V = v_ref[0]

    m_parts = []
    for b in range(_L // _QBLK):
        Qb = q_ref[0, b * _QBLK:(b + 1) * _QBLK, :]
        St = jax.lax.dot_general(K, Qb, (((1,), (1,)), ((), ())),
                                 preferred_element_type=jnp.float32)
        Cb = ct_ref[:, b * _QBLK:(b + 1) * _QBLK]
        mx = jnp.max(jnp.where(Cb > 0.0, St, -jnp.inf), axis=0, keepdims=True)
        sm = jnp.sum(St * Cb, axis=0, keepdims=True)
        m_parts.append(mx - sm * (1.0 / _L))
    M = jnp.concatenate(m_parts, axis=1)

    ids = jax.lax.broadcasted_iota(jnp.int32, (1, _L), 1)

    def tk(i, m_cur):
        m = jnp.max(m_cur)
        idx = jnp.min(jnp.where(m_cur == m, ids, _L))
        idx_ref[i] = idx
        qr_ref[pl.ds(i, 1), :] = q_ref[0, pl.ds(idx, 1), :]
        thr_ref[pl.ds(i, 1), :] = jnp.full((1, 1), idx, jnp.int32)
        return jnp.where(ids == idx, -jnp.inf, m_cur)

    jax.lax.fori_loop(0, _U, tk, M)

    Qr = qr_ref[...]
    scores = jax.lax.dot_general(Qr, K, (((1,), (1,)), ((), ())),
                                 preferred_element_type=jnp.float32)
    scores = scores * (1.0 / float(np.sqrt(_D)))
    kids = jax.lax.broadcasted_iota(jnp.int32, (_U, _L), 1)
    masked = jnp.where(kids > thr_ref[...], -jnp.inf, scores)
    mmax = jnp.max(masked, axis=1, keepdims=True)
    e = jnp.exp(masked - mmax)
    attn = e / jnp.sum(e, axis=1, keepdims=True)
    upd = jnp.dot(attn, V, preferred_element_type=jnp.float32)

    rr = jax.lax.broadcasted_iota(jnp.int32, (_CBLK, _CBLK), 0)
    cc = jax.lax.broadcasted_iota(jnp.int32, (_CBLK, _CBLK), 1)
    tri = (rr >= cc).astype(jnp.float32)

    def cs(j, carry):
        Vb = v_ref[0, pl.ds(j * _CBLK, _CBLK), :]
        blk = jnp.dot(tri, Vb, preferred_element_type=jnp.float32) + carry
        o_ref[0, pl.ds(j * _CBLK, _CBLK), :] = blk
        return blk[_CBLK - 1:_CBLK, :]

    jax.lax.fori_loop(0, _L // _CBLK, cs, jnp.zeros((1, _D), jnp.float32))

    for i in range(_U):
        idx = idx_ref[i]
        o_ref[0, pl.ds(idx, 1), :] = upd[i:i + 1, :]


def kernel(queries, keys, values):
    B, L, H, D = queries.shape
    Q = jnp.transpose(queries, (0, 2, 1, 3)).reshape(B * H, L, D)
    K = jnp.transpose(keys, (0, 2, 1, 3)).reshape(B * H, L, D)
    V = jnp.transpose(values, (0, 2, 1, 3)).reshape(B * H, L, D)
    ct = jnp.asarray(_CT_NP)

    out = pl.pallas_call(
        _body,
        grid=(B * H,),
        in_specs=[
            pl.BlockSpec((1, L, D), lambda i: (i, 0, 0)),
            pl.BlockSpec((1, L, D), lambda i: (i, 0, 0)),
            pl.BlockSpec((1, L, D), lambda i: (i, 0, 0)),
            pl.BlockSpec((L, L), lambda i: (0, 0)),
        ],
        out_specs=pl.BlockSpec((1, L, D), lambda i: (i, 0, 0)),
        out_shape=jax.ShapeDtypeStruct((B * H, L, D), jnp.float32),
        scratch_shapes=[
            pltpu.VMEM((_U, D), jnp.float32),
            pltpu.VMEM((_U, 1), jnp.int32),
            pltpu.SMEM((_U,), jnp.int32),
        ],
    )(Q, K, V, ct)

    return out.reshape(B, H, L, D)
